# Initial kernel scaffold; baseline (speedup 1.0000x reference)
import jax
import jax.numpy as jnp
from jax.experimental import pallas as pl


def kernel(hidden_states, input_ids, table):
    raise NotImplementedError("write your pallas kernel here")


# speedup vs baseline: 4.6100x; 4.6100x over previous
import functools

import jax
import jax.numpy as jnp
from jax import lax
from jax.experimental import pallas as pl
from jax.experimental.pallas import tpu as pltpu
from jax.experimental.pallas import tpu_sc as plsc

VOCAB = 50000
DIM = 64
NUM_HEADS = 4
B, L = 4, 8192
TOK = B * L

_M1 = (10007, 10009, 10037, 10039)
_M2 = (20011, 20021, 20023, 20029)
_C1 = tuple((256 * m) % VOCAB for m in _M1)
_C2 = tuple((256 * m) % VOCAB for m in _M2)

_LANES = 16
_NW = 32
_TPW = TOK // _NW
_CHUNK = 256
_NCHUNK = _TPW // _CHUNK
_IDXN = _CHUNK * NUM_HEADS
_STREAM = 128
_NSTREAM = _IDXN // _STREAM


def _lt(v, idx):
    return v.at[idx].get(mode="promise_in_bounds")


def _mod_v(y):
    q = (y.astype(jnp.float32) * (1.0 / VOCAB)).astype(jnp.int32)
    r = y - q * VOCAB
    r = jnp.where(r < 0, r + VOCAB, r)
    return jnp.where(r >= VOCAB, r - VOCAB, r)


@functools.partial(
    pl.kernel,
    mesh=plsc.VectorSubcoreMesh(core_axis_name="c", subcore_axis_name="s"),
    out_type=jax.ShapeDtypeStruct((TOK * NUM_HEADS, DIM), jnp.float32),
    scratch_types=[
        pltpu.VMEM((_CHUNK,), jnp.int32),
        pltpu.VMEM((_LANES,), jnp.int32),
        pltpu.VMEM((_IDXN,), jnp.int32),
        pltpu.VMEM((_IDXN, DIM), jnp.float32),
        pltpu.SemaphoreType.DMA,
    ],
    compiler_params=pltpu.CompilerParams(use_tc_tiling_on_sc=False),
)
def _engram_sc(flat_ref, table_ref, out_ref, win, pre, idxb, rows, sem):
    wid = lax.axis_index("s") * 2 + lax.axis_index("c")
    iota = lax.iota(jnp.int32, _LANES)
    shift_idx = jnp.maximum(iota - 1, 0)
    last_idx = iota * 0 + (_LANES - 1)
    head = iota & (NUM_HEADS - 1)

    def _headv(vals):
        v = jnp.full((_LANES,), vals[-1], dtype=jnp.int32)
        for k in range(NUM_HEADS - 2, -1, -1):
            v = jnp.where(head == k, vals[k], v)
        return v

    m1v = _headv(_M1)
    m2v = _headv(_M2)
    c1v = _headv(_C1)
    c2v = _headv(_C2)

    for c in range(_NCHUNK):
        base = wid * _TPW + c * _CHUNK
        pltpu.sync_copy(
            flat_ref.at[pl.ds(pl.multiple_of(base, 8), _CHUNK)], win
        )
        pltpu.sync_copy(
            flat_ref.at[
                pl.ds(pl.multiple_of(jnp.maximum(base - _LANES, 0), 8), _LANES)
            ],
            pre,
        )
        carry = _lt(pre[...], last_idx)

        for i in range(_CHUNK // _LANES):
            ids = win[pl.ds(i * _LANES, _LANES)]
            prev = jnp.where(iota == 0, carry, _lt(ids, shift_idx))
            carry = _lt(ids, last_idx)
            tokpos = iota + (base + i * _LANES)
            prev = jnp.where((tokpos & (L - 1)) == 0, 0, prev)
            for sub in range(_LANES // NUM_HEADS):
                tok = lax.shift_right_logical(iota, 2) + (sub * NUM_HEADS)
                idsx = _lt(ids, tok)
                prevx = _lt(prev, tok)
                y1 = lax.shift_right_logical(idsx, 8) * c1v + (idsx & 255) * m1v
                y2 = lax.shift_right_logical(prevx, 8) * c2v + (prevx & 255) * m2v
                h = _mod_v(y1) + _mod_v(y2)
                h = jnp.where(h >= VOCAB, h - VOCAB, h)
                idxb[pl.ds((i * NUM_HEADS + sub) * _LANES, _LANES)] = h

        copies = []
        for s in range(_NSTREAM):
            copies.append(
                pltpu.async_copy(
                    table_ref.at[idxb.at[pl.ds(s * _STREAM, _STREAM)]],
                    rows.at[pl.ds(s * _STREAM, _STREAM)],
                    sem,
                )
            )
        for cp in copies:
            cp.wait()

        pltpu.sync_copy(
            rows, out_ref.at[pl.ds(pl.multiple_of(base * NUM_HEADS, 8), _IDXN)]
        )


def kernel(hidden_states, input_ids, table):
    del hidden_states
    flat = input_ids.reshape(-1)
    out = _engram_sc(flat, table)
    return out.reshape(B, L, NUM_HEADS * DIM)


# speedup vs baseline: 4.6460x; 1.0078x over previous
import functools

import jax
import jax.numpy as jnp
from jax import lax
from jax.experimental import pallas as pl
from jax.experimental.pallas import tpu as pltpu
from jax.experimental.pallas import tpu_sc as plsc

VOCAB = 50000
DIM = 64
NUM_HEADS = 4
B, L = 4, 8192
TOK = B * L

_M1 = (10007, 10009, 10037, 10039)
_M2 = (20011, 20021, 20023, 20029)
_C1 = tuple((256 * m) % VOCAB for m in _M1)
_C2 = tuple((256 * m) % VOCAB for m in _M2)

_LANES = 16
_NW = 32
_TPW = TOK // _NW
_CHUNK = 256
_NCHUNK = _TPW // _CHUNK
_IDXN = _CHUNK * NUM_HEADS
_STREAM = 128
_NSTREAM = _IDXN // _STREAM


def _lt(v, idx):
    return v.at[idx].get(mode="promise_in_bounds")


def _mod_v(y):
    q = (y.astype(jnp.float32) * (1.0 / VOCAB)).astype(jnp.int32)
    r = y - q * VOCAB
    r = jnp.where(r < 0, r + VOCAB, r)
    return jnp.where(r >= VOCAB, r - VOCAB, r)


@functools.partial(
    pl.kernel,
    mesh=plsc.VectorSubcoreMesh(core_axis_name="c", subcore_axis_name="s"),
    out_type=jax.ShapeDtypeStruct((TOK * NUM_HEADS, DIM), jnp.float32),
    scratch_types=[
        pltpu.VMEM((_CHUNK,), jnp.int32),
        pltpu.VMEM((_LANES,), jnp.int32),
        pltpu.VMEM((_IDXN,), jnp.int32),
        pltpu.VMEM((_IDXN, DIM), jnp.float32),
        pltpu.SemaphoreType.DMA,
    ],
    compiler_params=pltpu.CompilerParams(use_tc_tiling_on_sc=False),
)
def _engram_sc(ids_ref, table_ref, out_ref, win, pre, idxb, rows, sem):
    wid = lax.axis_index("s") * 2 + lax.axis_index("c")
    row = lax.shift_right_logical(wid, 3)
    col0 = (wid & 7) * _TPW
    iota = lax.iota(jnp.int32, _LANES)
    shift_idx = jnp.maximum(iota - 1, 0)
    last_idx = iota * 0 + (_LANES - 1)
    head = iota & (NUM_HEADS - 1)

    def _headv(vals):
        v = jnp.full((_LANES,), vals[-1], dtype=jnp.int32)
        for k in range(NUM_HEADS - 2, -1, -1):
            v = jnp.where(head == k, vals[k], v)
        return v

    m1v = _headv(_M1)
    m2v = _headv(_M2)
    c1v = _headv(_C1)
    c2v = _headv(_C2)

    for c in range(_NCHUNK):
        base = wid * _TPW + c * _CHUNK
        col = col0 + c * _CHUNK
        pltpu.sync_copy(
            ids_ref.at[row, pl.ds(pl.multiple_of(col, 8), _CHUNK)], win
        )
        pltpu.sync_copy(
            ids_ref.at[
                row,
                pl.ds(pl.multiple_of(jnp.maximum(col - _LANES, 0), 8), _LANES),
            ],
            pre,
        )
        carry = _lt(pre[...], last_idx)

        for i in range(_CHUNK // _LANES):
            ids = win[pl.ds(i * _LANES, _LANES)]
            prev = jnp.where(iota == 0, carry, _lt(ids, shift_idx))
            carry = _lt(ids, last_idx)
            tokpos = iota + (base + i * _LANES)
            prev = jnp.where((tokpos & (L - 1)) == 0, 0, prev)
            for sub in range(_LANES // NUM_HEADS):
                tok = lax.shift_right_logical(iota, 2) + (sub * NUM_HEADS)
                idsx = _lt(ids, tok)
                prevx = _lt(prev, tok)
                y1 = lax.shift_right_logical(idsx, 8) * c1v + (idsx & 255) * m1v
                y2 = lax.shift_right_logical(prevx, 8) * c2v + (prevx & 255) * m2v
                h = _mod_v(y1) + _mod_v(y2)
                h = jnp.where(h >= VOCAB, h - VOCAB, h)
                idxb[pl.ds((i * NUM_HEADS + sub) * _LANES, _LANES)] = h

        copies = []
        for s in range(_NSTREAM):
            copies.append(
                pltpu.async_copy(
                    table_ref.at[idxb.at[pl.ds(s * _STREAM, _STREAM)]],
                    rows.at[pl.ds(s * _STREAM, _STREAM)],
                    sem,
                )
            )
        for cp in copies:
            cp.wait()

        pltpu.sync_copy(
            rows, out_ref.at[pl.ds(pl.multiple_of(base * NUM_HEADS, 8), _IDXN)]
        )


def kernel(hidden_states, input_ids, table):
    del hidden_states
    out = _engram_sc(input_ids, table)
    return out.reshape(B, L, NUM_HEADS * DIM)
